# baseline (speedup 1.0000x reference)
import functools

import jax
import jax.numpy as jnp
from jax import lax
from jax.experimental import pallas as pl
from jax.experimental.pallas import tpu as pltpu
from jax.experimental.pallas import tpu_sc as plsc

_NC, _NS = 2, 16
_NW = _NC * _NS
_RB = 32
_NCHUNK = 39


def _chunk(k):
    return k // 8, (k % 8) * _RB


def _sc_body(y_hbm, out_hbm, buf0, buf1, buf2, psrc, pavg,
             s_in0, s_in1, s_in2, s_out0, s_out1, s_out2, s_p, s_pout):
    wid = lax.axis_index("s") * _NC + lax.axis_index("c")
    bufs = (buf0, buf1, buf2)
    s_ins = (s_in0, s_in1, s_in2)
    s_outs = (s_out0, s_out1, s_out2)
    loads = [None] * 3
    stores = [None] * 3

    p0, r0 = _chunk(wid)

    is_patch = wid < 24

    @pl.when(is_patch)
    def _():
        for q in range(3):
            pltpu.async_copy(
                y_hbm.at[p0 + 3 * (q + 1), pl.ds(r0, _RB), :],
                psrc.at[q], s_p)

    @pl.when(jnp.logical_not(is_patch))
    def _():
        pltpu.async_copy(y_hbm.at[p0, pl.ds(r0, _RB), :], psrc.at[0], s_p)

    for t in (1, 2):
        k = wid + _NW * t
        p, r = _chunk(k)
        loads[t % 3] = pltpu.async_copy(
            y_hbm.at[p, pl.ds(r, _RB), :], bufs[t % 3], s_ins[t % 3])

    @pl.when(is_patch)
    def _():
        pltpu.make_async_copy(
            y_hbm.at[0, pl.ds(0, _RB), :], psrc, s_p).wait()

        def _mean_step(t, carry):
            r = t // 32
            l = (t % 32) * 16
            pavg[r, pl.ds(l, 16)] = (
                psrc[0, r, pl.ds(l, 16)] + psrc[1, r, pl.ds(l, 16)]
                + psrc[2, r, pl.ds(l, 16)]) * (1.0 / 3.0)
            return carry

        lax.fori_loop(0, _RB * 32, _mean_step, 0)
        pltpu.async_copy(pavg, out_hbm.at[p0, pl.ds(r0, _RB), :], s_pout)

    @pl.when(jnp.logical_not(is_patch))
    def _():
        pltpu.make_async_copy(
            y_hbm.at[0, pl.ds(0, _RB), :], psrc.at[0], s_p).wait()
        pltpu.async_copy(psrc.at[0], out_hbm.at[p0, pl.ds(r0, _RB), :],
                         s_pout)

    for t in range(1, _NCHUNK):
        i = t % 3
        k = wid + _NW * t
        p, r = _chunk(k)
        loads[i].wait()
        stores[i] = pltpu.async_copy(
            bufs[i], out_hbm.at[p, pl.ds(r, _RB), :], s_outs[i])
        tn = t + 2
        if tn < _NCHUNK:
            j = tn % 3
            if stores[j] is not None:
                stores[j].wait()
            kn = wid + _NW * tn
            pn, rn = _chunk(kn)
            loads[j] = pltpu.async_copy(
                y_hbm.at[pn, pl.ds(rn, _RB), :], bufs[j], s_ins[j])

    stores[(_NCHUNK - 1) % 3].wait()
    stores[(_NCHUNK - 2) % 3].wait()
    stores[(_NCHUNK - 3) % 3].wait()
    pltpu.make_async_copy(
        y_hbm.at[0, pl.ds(0, _RB), :], pavg, s_pout).wait()


def _make_sc_call(planes, B, F):
    mesh = plsc.VectorSubcoreMesh(core_axis_name="c", subcore_axis_name="s")
    return functools.partial(
        pl.kernel,
        out_type=jax.ShapeDtypeStruct((planes, B, F), jnp.float32),
        mesh=mesh,
        scratch_types=[
            pltpu.VMEM((_RB, F), jnp.float32),
            pltpu.VMEM((_RB, F), jnp.float32),
            pltpu.VMEM((_RB, F), jnp.float32),
            pltpu.VMEM((3, _RB, F), jnp.float32),
            pltpu.VMEM((_RB, F), jnp.float32),
            pltpu.SemaphoreType.DMA,
            pltpu.SemaphoreType.DMA,
            pltpu.SemaphoreType.DMA,
            pltpu.SemaphoreType.DMA,
            pltpu.SemaphoreType.DMA,
            pltpu.SemaphoreType.DMA,
            pltpu.SemaphoreType.DMA,
            pltpu.SemaphoreType.DMA,
        ],
    )(_sc_body)


def kernel(x):
    B, F, J, C = x.shape
    planes = J * C
    y = jnp.transpose(x, (2, 3, 0, 1)).reshape(planes, B, F)
    out = _make_sc_call(planes, B, F)(y)
    return jnp.transpose(out.reshape(J, C, B, F), (2, 3, 0, 1))


# speedup vs baseline: 1.1652x; 1.1652x over previous
import functools

import jax
import jax.numpy as jnp
from jax import lax
from jax.experimental import pallas as pl
from jax.experimental.pallas import tpu as pltpu
from jax.experimental.pallas import tpu_sc as plsc

_NC, _NS = 2, 16
_GP = 9
_TC_BLK = 12


def _sc_body(y_hbm, out_hbm, buf, avg, s_in, s_out, s_avg):
    wid = lax.axis_index("s") * _NC + lax.axis_index("c")
    r0 = wid * 8
    pltpu.async_copy(
        y_hbm.at[pl.ds(3, _GP), pl.ds(r0, 8), :], buf, s_in).wait()
    store = pltpu.async_copy(
        buf, out_hbm.at[pl.ds(3, _GP), pl.ds(r0, 8), :], s_out)

    def _mean_step(t, carry):
        r = t // 32
        l = (t % 32) * 16
        for c in range(3):
            v = (buf[c, r, pl.ds(l, 16)]
                 + buf[c + 3, r, pl.ds(l, 16)]
                 + buf[c + 6, r, pl.ds(l, 16)]) * (1.0 / 3.0)
            avg[c, r, pl.ds(l, 16)] = v
        return carry

    lax.fori_loop(0, 8 * 32, _mean_step, 0)
    pltpu.async_copy(
        avg, out_hbm.at[pl.ds(0, 3), pl.ds(r0, 8), :], s_avg).wait()
    store.wait()


def _make_sc_call(planes, B, F):
    mesh = plsc.VectorSubcoreMesh(core_axis_name="c", subcore_axis_name="s")
    return functools.partial(
        pl.kernel,
        out_type=jax.ShapeDtypeStruct((planes, B, F), jnp.float32),
        mesh=mesh,
        scratch_types=[
            pltpu.VMEM((_GP, 8, F), jnp.float32),
            pltpu.VMEM((3, 8, F), jnp.float32),
            pltpu.SemaphoreType.DMA,
            pltpu.SemaphoreType.DMA,
            pltpu.SemaphoreType.DMA,
        ],
    )(_sc_body)


def _tc_body(x_ref, alias_ref, o_ref):
    del alias_ref
    o_ref[...] = x_ref[...]


def kernel(x):
    B, F, J, C = x.shape
    planes = J * C
    y = jnp.transpose(x, (2, 3, 0, 1)).reshape(planes, B, F)
    out0 = _make_sc_call(planes, B, F)(y)
    out = pl.pallas_call(
        _tc_body,
        grid=((planes - 12) // _TC_BLK,),
        in_specs=[
            pl.BlockSpec((_TC_BLK, B, F), lambda i: (i + 1, 0, 0)),
            pl.BlockSpec(memory_space=pl.ANY),
        ],
        out_specs=pl.BlockSpec((_TC_BLK, B, F), lambda i: (i + 1, 0, 0)),
        out_shape=jax.ShapeDtypeStruct((planes, B, F), x.dtype),
        input_output_aliases={1: 0},
    )(y, out0)
    return jnp.transpose(out.reshape(J, C, B, F), (2, 3, 0, 1))
